# baseline (speedup 1.0000x reference)
import functools

import jax
import jax.numpy as jnp
from jax import lax
from jax.experimental import pallas as pl
from jax.experimental.pallas import tpu as pltpu
from jax.experimental.pallas import tpu_sc as plsc

_NC, _NS = 2, 16
_NW = _NC * _NS
_CHUNK = 800


@functools.cache
def _build_gather(n, d):
    n_per_w = n // _NW
    n_chunks = n_per_w // _CHUNK
    assert n_chunks % 2 == 0 and n_chunks >= 4
    mesh = plsc.VectorSubcoreMesh(core_axis_name="c", subcore_axis_name="s")

    @functools.partial(
        pl.kernel,
        out_type=jax.ShapeDtypeStruct((n, d), jnp.float32),
        mesh=mesh,
        scratch_types=[
            pltpu.VMEM((_CHUNK,), jnp.int32),
            pltpu.VMEM((_CHUNK,), jnp.int32),
            pltpu.VMEM((_CHUNK, d), jnp.float32),
            pltpu.VMEM((_CHUNK, d), jnp.float32),
            pltpu.SemaphoreType.DMA,
            pltpu.SemaphoreType.DMA,
            pltpu.SemaphoreType.DMA,
            pltpu.SemaphoreType.DMA,
            pltpu.SemaphoreType.DMA,
            pltpu.SemaphoreType.DMA,
        ],
        compiler_params=pltpu.CompilerParams(use_tc_tiling_on_sc=False),
    )
    def gather(idx_hbm, table_hbm, out_hbm,
               idx0, idx1, rows0, rows1,
               isem0, isem1, gsem0, gsem1, wsem0, wsem1):
        wid = lax.axis_index("s") * _NC + lax.axis_index("c")
        base = wid * n_per_w
        idx_v = (idx0, idx1)
        rows_v = (rows0, rows1)
        isem = (isem0, isem1)
        gsem = (gsem0, gsem1)
        wsem = (wsem0, wsem1)

        def idx_start(i, u):
            pltpu.async_copy(
                idx_hbm.at[pl.ds(base + i * _CHUNK, _CHUNK)], idx_v[u],
                isem[u])

        def idx_wait(u):
            pltpu.make_async_copy(
                idx_hbm.at[pl.ds(0, _CHUNK)], idx_v[u], isem[u]).wait()

        def gather_start(u):
            pltpu.async_copy(table_hbm.at[idx_v[u]], rows_v[u], gsem[u])

        def gather_wait(u):
            pltpu.make_async_copy(
                table_hbm.at[idx_v[u]], rows_v[u], gsem[u]).wait()

        def write_start(i, u):
            pltpu.async_copy(
                rows_v[u], out_hbm.at[pl.ds(base + i * _CHUNK, _CHUNK)],
                wsem[u])

        def write_wait(u):
            pltpu.make_async_copy(
                rows_v[u], out_hbm.at[pl.ds(0, _CHUNK)], wsem[u]).wait()

        idx_start(0, 0)
        idx_start(1, 1)
        idx_wait(0)
        gather_start(0)

        @pl.loop(0, n_chunks // 2)
        def _outer(j):
            for u in (0, 1):
                i = j * 2 + u
                nu = 1 - u
                gather_wait(u)

                @pl.when(i + 2 < n_chunks)
                def _():
                    idx_start(i + 2, u)

                @pl.when(i + 1 < n_chunks)
                def _():
                    idx_wait(nu)

                    @pl.when(i >= 1)
                    def _():
                        write_wait(nu)
                    gather_start(nu)

                write_start(i, u)

        write_wait(0)
        write_wait(1)

    return gather


def kernel(sequences_tokens, embedding_table):
    b, l = sequences_tokens.shape
    v, d = embedding_table.shape
    idx = sequences_tokens.reshape(b * l)
    av = jnp.arange(2)[:, None, None]
    bv = jnp.arange(d)[None, :, None]
    cv = jnp.arange(2 * d)[None, None, :]
    pack_onehot = (cv == av * d + bv).astype(jnp.float32)
    t2 = lax.dot_general(
        embedding_table.reshape(v // 2, 2, d), pack_onehot,
        (((1, 2), (0, 1)), ((), ())),
        preferred_element_type=jnp.float32)
    t_view = t2.reshape(v, d)
    out = _build_gather(b * l, d)(idx, t_view)
    return out.reshape(b, l, d)
